# Initial kernel scaffold; baseline (speedup 1.0000x reference)
import jax
import jax.numpy as jnp
from jax.experimental import pallas as pl


def kernel(x):
    raise NotImplementedError("write your pallas kernel here")


# speedup vs baseline: 4.5755x; 4.5755x over previous
import jax
import jax.numpy as jnp
from jax.experimental import pallas as pl
from jax.experimental.pallas import tpu as pltpu

_BLK = 4


def _var3x3_body(x_ref, o_ref):
    x = x_ref[...]
    x2 = x * x

    zc = jnp.zeros_like(x[:, :, :1])

    def hsum(a):
        left = jnp.concatenate([zc, a[:, :, :-1]], axis=2)
        right = jnp.concatenate([a[:, :, 1:], zc], axis=2)
        return left + a + right

    h1 = hsum(x)
    h2 = hsum(x2)

    zr = jnp.zeros_like(x[:, :1, :])

    def vsum(a):
        up = jnp.concatenate([zr, a[:, :-1, :]], axis=1)
        down = jnp.concatenate([a[:, 1:, :], zr], axis=1)
        return up + a + down

    s1 = vsum(h1)
    s2 = vsum(h2)

    inv9 = jnp.float32(1.0 / 9.0)
    m = s1 * inv9
    o_ref[...] = s2 * inv9 - m * m


def kernel(x):
    B, C, H, W = x.shape
    n = B * C
    xr = x.reshape(n, H, W)
    out = pl.pallas_call(
        _var3x3_body,
        grid=(n // _BLK,),
        in_specs=[pl.BlockSpec((_BLK, H, W), lambda i: (i, 0, 0))],
        out_specs=pl.BlockSpec((_BLK, H, W), lambda i: (i, 0, 0)),
        out_shape=jax.ShapeDtypeStruct((n, H, W), x.dtype),
        compiler_params=pltpu.CompilerParams(
            dimension_semantics=("parallel",),
        ),
    )(xr)
    return out.reshape(B, C, H, W)


# speedup vs baseline: 5.5772x; 1.2189x over previous
import jax
import jax.numpy as jnp
from jax.experimental import pallas as pl
from jax.experimental.pallas import tpu as pltpu

_BLK = 4


def _var3x3_body(x_ref, o_ref):
    x = x_ref[...]

    zr = jnp.zeros_like(x[:, :1, :])
    u = jnp.concatenate([zr, x[:, :-1, :]], axis=1)
    d = jnp.concatenate([x[:, 1:, :], zr], axis=1)
    v1 = u + x + d
    v2 = u * u + x * x + d * d

    zc = jnp.zeros_like(x[:, :, :1])

    def hsum(a):
        left = jnp.concatenate([zc, a[:, :, :-1]], axis=2)
        right = jnp.concatenate([a[:, :, 1:], zc], axis=2)
        return left + a + right

    s1 = hsum(v1)
    s2 = hsum(v2)

    inv9 = jnp.float32(1.0 / 9.0)
    m = s1 * inv9
    o_ref[...] = s2 * inv9 - m * m


def kernel(x):
    B, C, H, W = x.shape
    n = B * C
    xr = x.reshape(n, H, W)
    out = pl.pallas_call(
        _var3x3_body,
        grid=(n // _BLK,),
        in_specs=[pl.BlockSpec((_BLK, H, W), lambda i: (i, 0, 0))],
        out_specs=pl.BlockSpec((_BLK, H, W), lambda i: (i, 0, 0)),
        out_shape=jax.ShapeDtypeStruct((n, H, W), x.dtype),
        compiler_params=pltpu.CompilerParams(
            dimension_semantics=("parallel",),
        ),
    )(xr)
    return out.reshape(B, C, H, W)
